# baseline (speedup 1.0000x reference)
import functools

import jax
import jax.numpy as jnp
import numpy as np
from jax import lax
from jax.experimental import pallas as pl
from jax.experimental.pallas import tpu as pltpu
from jax.experimental.pallas import tpu_sc as plsc

N_NODES = 50000
FEAT = 128
EMBED = 128
S = 10

NC = 2
NS = 16
NW = NC * NS

B_PAD = 50176
BPW = B_PAD // NW
NCK = 32
CHUNKS = BPW // NCK
ROWS = (S + 1) * NCK


SLICES = ((0, 128), (128, 128), (256, ROWS - 256))


def _sc_body(nodes_hbm, tab_hbm, feat_hbm, self_out, sum_out,
             nodes_v, tn0, tn1, gath0, gath1, acc0, acc1, sw0, sw1,
             semA, semI, semW0, semW1):
    wid = lax.axis_index("s") * NC + lax.axis_index("c")
    base = wid * BPW
    tn_v = (tn0, tn1)
    gath_v = (gath0, gath1)
    acc_v = (acc0, acc1)
    sw_v = (sw0, sw1)
    semW = (semW0, semW1)


    def ids_descs(c, p):
        cb = pl.multiple_of(c * NCK, 8)
        return [pltpu.make_async_copy(
                    tab_hbm.at[pl.ds(j * N_NODES, N_NODES)]
                           .at[nodes_v.at[pl.ds(cb, NCK)]],
                    tn_v[p].at[pl.ds(j * NCK, NCK)], semI)
                for j in range(S)]

    def self_ids(c, p):
        cb = pl.multiple_of(c * NCK, 8)
        for h in range(NCK // 16):
            tn_v[p][pl.ds(S * NCK + h * 16, 16)] = (
                nodes_v[pl.ds(cb + h * 16, 16)])

    def feat_descs(p):
        return [pltpu.make_async_copy(feat_hbm.at[tn_v[p]], gath_v[p], semA)]

    def write_descs(c, p):
        dst = base + c * NCK
        return [pltpu.make_async_copy(sw_v[p],
                                      self_out.at[pl.ds(dst, NCK)], semW[p]),
                pltpu.make_async_copy(acc_v[p],
                                      sum_out.at[pl.ds(dst, NCK)], semW[p])]

    pltpu.sync_copy(nodes_hbm.at[pl.ds(base, BPW)], nodes_v)
    for d in ids_descs(0, 0):
        d.start()
    self_ids(0, 0)
    for d in ids_descs(0, 0):
        d.wait()
    for d in feat_descs(0):
        d.start()
    for d in ids_descs(1, 1):
        d.start()
    self_ids(1, 1)

    def do_iter(c, p):
        for d in feat_descs(p):
            d.wait()

        @pl.when(c + 1 < CHUNKS)
        def _():
            for d in ids_descs(c + 1, 1 - p):
                d.wait()

            @pl.when(c >= 1)
            def _():
                for d in write_descs(c - 1, 1 - p):
                    d.wait()

            for d in feat_descs(1 - p):
                d.start()

            @pl.when(c + 2 < CHUNKS)
            def _():
                for d in ids_descs(c + 2, p):
                    d.start()
                self_ids(c + 2, p)

        gw = gath_v[p]
        aw = acc_v[p]
        fmt = plsc.PackFormat.INTERLEAVED

        def pack_words(a, b):
            return plsc.bitcast(plsc.pack(a, b, format=fmt), jnp.int32)

        def red_row(r, _):
            for g in range(FEAT // 32):
                a = gw[r, pl.ds(g * 32, 16)]
                b = gw[r, pl.ds(g * 32 + 16, 16)]
                for j in range(1, S):
                    a = a + gw[j * NCK + r, pl.ds(g * 32, 16)]
                    b = b + gw[j * NCK + r, pl.ds(g * 32 + 16, 16)]
                aw[r, pl.ds(g * 16, 16)] = pack_words(a, b)
            return 0

        lax.fori_loop(0, NCK, red_row, 0)

        for r in range(NCK):
            for g in range(FEAT // 32):
                sw_v[p][r, pl.ds(g * 16, 16)] = pack_words(
                    gw[S * NCK + r, pl.ds(g * 32, 16)],
                    gw[S * NCK + r, pl.ds(g * 32 + 16, 16)])

        for d in write_descs(c, p):
            d.start()

    def chunk(c, _):
        @pl.when(c % 2 == 0)
        def _():
            do_iter(c, 0)

        @pl.when(c % 2 == 1)
        def _():
            do_iter(c, 1)

        return 0

    lax.fori_loop(0, CHUNKS, chunk, 0)

    for d in write_descs(CHUNKS - 2, (CHUNKS - 2) % 2):
        d.wait()
    for d in write_descs(CHUNKS - 1, (CHUNKS - 1) % 2):
        d.wait()


@functools.partial(
    pl.kernel,
    out_type=(jax.ShapeDtypeStruct((B_PAD, FEAT // 2), jnp.int32),
              jax.ShapeDtypeStruct((B_PAD, FEAT // 2), jnp.int32)),
    mesh=plsc.VectorSubcoreMesh(core_axis_name="c", subcore_axis_name="s"),
    compiler_params=pltpu.CompilerParams(needs_layout_passes=False),
    scratch_types=[
        pltpu.VMEM((BPW,), jnp.int32),
        pltpu.VMEM((ROWS,), jnp.int32),
        pltpu.VMEM((ROWS,), jnp.int32),
        pltpu.VMEM((ROWS, FEAT), jnp.float32),
        pltpu.VMEM((ROWS, FEAT), jnp.float32),
        pltpu.VMEM((NCK, FEAT // 2), jnp.int32),
        pltpu.VMEM((NCK, FEAT // 2), jnp.int32),
        pltpu.VMEM((NCK, FEAT // 2), jnp.int32),
        pltpu.VMEM((NCK, FEAT // 2), jnp.int32),
        pltpu.SemaphoreType.DMA,
        pltpu.SemaphoreType.DMA,
        pltpu.SemaphoreType.DMA,
        pltpu.SemaphoreType.DMA,
    ],
)
def _sc_gather(*refs):
    _sc_body(*refs)


_LO = (np.arange(FEAT // 2) // 16) * 32 + np.arange(FEAT // 2) % 16
_HI = _LO + 16


def _bf16_pair_to_f32(w):
    lo = lax.bitcast_convert_type(lax.shift_left(w, 16), jnp.float32)
    hi = lax.bitcast_convert_type(
        lax.bitwise_and(w, jnp.int32(-65536)), jnp.float32)
    return lo, hi


def _tc_body(s_ref, n_ref, w1l_ref, w1h_ref, w2l_ref, w2h_ref, out_ref):
    dn = (((1,), (1,)), ((), ()))
    se, so = _bf16_pair_to_f32(s_ref[...])
    ne, no = _bf16_pair_to_f32(n_ref[...])
    acc = lax.dot_general(se, w1l_ref[...], dn,
                          preferred_element_type=jnp.float32)
    acc += lax.dot_general(so, w1h_ref[...], dn,
                           preferred_element_type=jnp.float32)
    acc += lax.dot_general(ne, w2l_ref[...], dn,
                           preferred_element_type=jnp.float32)
    acc += lax.dot_general(no, w2h_ref[...], dn,
                           preferred_element_type=jnp.float32)
    out_ref[...] = jnp.maximum(acc, 0.0)


def _tc_matmul(self_f, sum_f, w1l, w1h, w2l, w2h):
    bt = 1024
    grid = B_PAD // bt
    h = FEAT // 2
    return pl.pallas_call(
        _tc_body,
        grid=(grid,),
        in_specs=[
            pl.BlockSpec((bt, h), lambda i: (i, 0)),
            pl.BlockSpec((bt, h), lambda i: (i, 0)),
            pl.BlockSpec((EMBED, h), lambda i: (0, 0)),
            pl.BlockSpec((EMBED, h), lambda i: (0, 0)),
            pl.BlockSpec((EMBED, h), lambda i: (0, 0)),
            pl.BlockSpec((EMBED, h), lambda i: (0, 0)),
        ],
        out_specs=pl.BlockSpec((bt, EMBED), lambda i: (i, 0)),
        out_shape=jax.ShapeDtypeStruct((N_NODES, EMBED), jnp.float32),
    )(self_f, sum_f, w1l, w1h, w2l, w2h)


def kernel(nodes, features, neigh_idx, weight):
    nodes_p = jnp.concatenate(
        [nodes.astype(jnp.int32),
         jnp.zeros((B_PAD - N_NODES,), jnp.int32)])
    tab = neigh_idx.astype(jnp.int32).T.reshape(-1)
    self_f, sum_f = _sc_gather(nodes_p, tab, features)
    w1 = weight[:, :FEAT]
    w2s = weight[:, FEAT:] * (1.0 / S)
    lo = jnp.asarray(_LO)
    hi = jnp.asarray(_HI)
    return _tc_matmul(self_f, sum_f, w1[:, lo], w1[:, hi],
                      w2s[:, lo], w2s[:, hi]).T


# speedup vs baseline: 1.0972x; 1.0972x over previous
import functools

import jax
import jax.numpy as jnp
import numpy as np
from jax import lax
from jax.experimental import pallas as pl
from jax.experimental.pallas import tpu as pltpu
from jax.experimental.pallas import tpu_sc as plsc

N_NODES = 50000
FEAT = 128
EMBED = 128
S = 10

NC = 2
NS = 16
NW = NC * NS

B_PAD = 50176
BPW = B_PAD // NW
NCK = 32
CHUNKS = BPW // NCK
ROWS = (S + 1) * NCK


SLICES = ((0, 128), (128, 128), (256, ROWS - 256))


def _sc_body(nodes_hbm, tab_hbm, feat_hbm, self_out, sum_out,
             nodes_v, tn0, tn1, gath0, gath1, acc0, acc1, sw0, sw1,
             semA, semI, semW0, semW1):
    wid = lax.axis_index("s") * NC + lax.axis_index("c")
    base = wid * BPW
    tn_v = (tn0, tn1)
    gath_v = (gath0, gath1)
    acc_v = (acc0, acc1)
    sw_v = (sw0, sw1)
    semW = (semW0, semW1)


    def ids_descs(c, p):
        cb = pl.multiple_of(c * NCK, 8)
        return [pltpu.make_async_copy(
                    tab_hbm.at[pl.ds(j * N_NODES, N_NODES)]
                           .at[nodes_v.at[pl.ds(cb, NCK)]],
                    tn_v[p].at[pl.ds(j * NCK, NCK)], semI)
                for j in range(S)]

    def self_ids(c, p):
        cb = pl.multiple_of(c * NCK, 8)
        for h in range(NCK // 16):
            tn_v[p][pl.ds(S * NCK + h * 16, 16)] = (
                nodes_v[pl.ds(cb + h * 16, 16)])

    def feat_descs(p):
        return [pltpu.make_async_copy(feat_hbm.at[tn_v[p]], gath_v[p], semA)]

    def write_descs(c, p):
        dst = base + c * NCK
        return [pltpu.make_async_copy(sw_v[p],
                                      self_out.at[pl.ds(dst, NCK)], semW[p]),
                pltpu.make_async_copy(acc_v[p],
                                      sum_out.at[pl.ds(dst, NCK)], semW[p])]

    pltpu.sync_copy(nodes_hbm.at[pl.ds(base, BPW)], nodes_v)
    for d in ids_descs(0, 0):
        d.start()
    self_ids(0, 0)
    for d in ids_descs(0, 0):
        d.wait()
    for d in feat_descs(0):
        d.start()
    for d in ids_descs(1, 1):
        d.start()
    self_ids(1, 1)

    def do_iter(c, p):
        for d in feat_descs(p):
            d.wait()

        @pl.when(c + 1 < CHUNKS)
        def _():
            for d in ids_descs(c + 1, 1 - p):
                d.wait()

            @pl.when(c >= 1)
            def _():
                for d in write_descs(c - 1, 1 - p):
                    d.wait()

            for d in feat_descs(1 - p):
                d.start()

            @pl.when(c + 2 < CHUNKS)
            def _():
                for d in ids_descs(c + 2, p):
                    d.start()
                self_ids(c + 2, p)

        gw = gath_v[p]
        aw = acc_v[p]
        fmt = plsc.PackFormat.INTERLEAVED

        def pack_words(a, b):
            return plsc.bitcast(plsc.pack(a, b, format=fmt), jnp.int32)

        def red_row(r, _):
            for g in range(FEAT // 32):
                a = gw[r, pl.ds(g * 32, 16)]
                b = gw[r, pl.ds(g * 32 + 16, 16)]
                for j in range(1, S):
                    a = a + gw[j * NCK + r, pl.ds(g * 32, 16)]
                    b = b + gw[j * NCK + r, pl.ds(g * 32 + 16, 16)]
                aw[r, pl.ds(g * 16, 16)] = pack_words(a, b)
            return 0

        lax.fori_loop(0, NCK, red_row, 0)

        for r in range(NCK):
            for g in range(FEAT // 32):
                sw_v[p][r, pl.ds(g * 16, 16)] = pack_words(
                    gw[S * NCK + r, pl.ds(g * 32, 16)],
                    gw[S * NCK + r, pl.ds(g * 32 + 16, 16)])

        for d in write_descs(c, p):
            d.start()

    def chunk(c, _):
        @pl.when(c % 2 == 0)
        def _():
            do_iter(c, 0)

        @pl.when(c % 2 == 1)
        def _():
            do_iter(c, 1)

        return 0

    lax.fori_loop(0, CHUNKS, chunk, 0)

    for d in write_descs(CHUNKS - 2, (CHUNKS - 2) % 2):
        d.wait()
    for d in write_descs(CHUNKS - 1, (CHUNKS - 1) % 2):
        d.wait()


@functools.partial(
    pl.kernel,
    out_type=(jax.ShapeDtypeStruct((B_PAD, FEAT // 2), jnp.int32),
              jax.ShapeDtypeStruct((B_PAD, FEAT // 2), jnp.int32)),
    mesh=plsc.VectorSubcoreMesh(core_axis_name="c", subcore_axis_name="s"),
    compiler_params=pltpu.CompilerParams(needs_layout_passes=False),
    scratch_types=[
        pltpu.VMEM((BPW,), jnp.int32),
        pltpu.VMEM((ROWS,), jnp.int32),
        pltpu.VMEM((ROWS,), jnp.int32),
        pltpu.VMEM((ROWS, FEAT), jnp.float32),
        pltpu.VMEM((ROWS, FEAT), jnp.float32),
        pltpu.VMEM((NCK, FEAT // 2), jnp.int32),
        pltpu.VMEM((NCK, FEAT // 2), jnp.int32),
        pltpu.VMEM((NCK, FEAT // 2), jnp.int32),
        pltpu.VMEM((NCK, FEAT // 2), jnp.int32),
        pltpu.SemaphoreType.DMA,
        pltpu.SemaphoreType.DMA,
        pltpu.SemaphoreType.DMA,
        pltpu.SemaphoreType.DMA,
    ],
)
def _sc_gather(*refs):
    _sc_body(*refs)


_LO = (np.arange(FEAT // 2) // 16) * 32 + np.arange(FEAT // 2) % 16
_HI = _LO + 16


def _bf16_pair_to_f32(w):
    lo = lax.bitcast_convert_type(lax.shift_left(w, 16), jnp.float32)
    hi = lax.bitcast_convert_type(
        lax.bitwise_and(w, jnp.int32(-65536)), jnp.float32)
    return lo, hi


def _tc_body(s_ref, n_ref, w1l_ref, w1h_ref, w2l_ref, w2h_ref, out_ref):
    dn = (((1,), (1,)), ((), ()))
    se, so = _bf16_pair_to_f32(s_ref[...])
    ne, no = _bf16_pair_to_f32(n_ref[...])
    acc = lax.dot_general(se, w1l_ref[...], dn,
                          preferred_element_type=jnp.float32)
    acc += lax.dot_general(so, w1h_ref[...], dn,
                           preferred_element_type=jnp.float32)
    acc += lax.dot_general(ne, w2l_ref[...], dn,
                           preferred_element_type=jnp.float32)
    acc += lax.dot_general(no, w2h_ref[...], dn,
                           preferred_element_type=jnp.float32)
    out_ref[...] = jnp.maximum(acc, 0.0)


def _tc_matmul(self_f, sum_f, w1l, w1h, w2l, w2h):
    bt = 3584
    grid = B_PAD // bt
    h = FEAT // 2
    return pl.pallas_call(
        _tc_body,
        grid=(grid,),
        in_specs=[
            pl.BlockSpec((bt, h), lambda i: (i, 0)),
            pl.BlockSpec((bt, h), lambda i: (i, 0)),
            pl.BlockSpec((EMBED, h), lambda i: (0, 0)),
            pl.BlockSpec((EMBED, h), lambda i: (0, 0)),
            pl.BlockSpec((EMBED, h), lambda i: (0, 0)),
            pl.BlockSpec((EMBED, h), lambda i: (0, 0)),
        ],
        out_specs=pl.BlockSpec((bt, EMBED), lambda i: (i, 0)),
        out_shape=jax.ShapeDtypeStruct((N_NODES, EMBED), jnp.float32),
    )(self_f, sum_f, w1l, w1h, w2l, w2h)


def kernel(nodes, features, neigh_idx, weight):
    nodes_p = jnp.concatenate(
        [nodes.astype(jnp.int32),
         jnp.zeros((B_PAD - N_NODES,), jnp.int32)])
    tab = neigh_idx.astype(jnp.int32).T.reshape(-1)
    self_f, sum_f = _sc_gather(nodes_p, tab, features)
    w1 = weight[:, :FEAT]
    w2s = weight[:, FEAT:] * (1.0 / S)
    lo = jnp.asarray(_LO)
    hi = jnp.asarray(_HI)
    return _tc_matmul(self_f, sum_f, w1[:, lo], w1[:, hi],
                      w2s[:, lo], w2s[:, hi]).T
